# Initial kernel scaffold; baseline (speedup 1.0000x reference)
import jax
import jax.numpy as jnp
from jax.experimental import pallas as pl


def kernel(x, edge_index, W1, b1, W2, b2, WL1, bL1, WL2, bL2):
    raise NotImplementedError("write your pallas kernel here")


# speedup vs baseline: 22.5106x; 22.5106x over previous
import functools

import jax
import jax.numpy as jnp
from jax import lax
from jax.experimental import pallas as pl
from jax.experimental.pallas import tpu as pltpu
from jax.experimental.pallas import tpu_sc as plsc

N_NODES = 100000
N_EDGES = 1600000
LANES = 16
IDX_W = 128
K_PER_CHUNK = 8
CHUNK = IDX_W * K_PER_CHUNK
EDGES_PAD = 98 * 16 * CHUNK
IDX_ROWS = EDGES_PAD // IDX_W
CONV_CHUNKS = EDGES_PAD // (16 * CHUNK)
DEG_CHUNKS = EDGES_PAD // (32 * CHUNK)
ACC_ROWS = 100096
ZERO_ROWS = ACC_ROWS // 16
BLK = 1000
GRID = N_NODES // BLK


def _sc_mesh():
    return plsc.VectorSubcoreMesh(core_axis_name="c", subcore_axis_name="s")


def _deg_pass(dstp, ones_h, zeros_h):
    @functools.partial(
        pl.kernel,
        out_type=(
            jax.ShapeDtypeStruct((ACC_ROWS, LANES), jnp.float32),
            jax.ShapeDtypeStruct((ACC_ROWS, LANES), jnp.float32),
        ),
        mesh=_sc_mesh(),
        compiler_params=pltpu.CompilerParams(use_tc_tiling_on_sc=False),
        scratch_types=[
            pltpu.VMEM((K_PER_CHUNK, IDX_W), jnp.int32),
            pltpu.VMEM((IDX_W, LANES), jnp.float32),
            pltpu.VMEM_SHARED((ACC_ROWS, LANES), jnp.float32),
        ],
    )
    def kdeg(dstp_h, ones_hbm, zeros_hbm, out0_h, out1_h, didx, ones_v, acc):
        c = lax.axis_index("c")
        s = lax.axis_index("s")
        pltpu.sync_copy(zeros_hbm, acc.at[pl.ds(s * ZERO_ROWS, ZERO_ROWS)])
        pltpu.sync_copy(ones_hbm, ones_v)
        plsc.subcore_barrier()
        wid = s * 2 + c
        base = wid * (DEG_CHUNKS * K_PER_CHUNK)

        def body(i, carry):
            rb = base + i * K_PER_CHUNK
            pltpu.sync_copy(dstp_h.at[pl.ds(rb, K_PER_CHUNK)], didx)
            for j in range(K_PER_CHUNK):
                pltpu.sync_copy(ones_v, acc.at[didx.at[j]], add=True)
            return carry

        lax.fori_loop(0, DEG_CHUNKS, body, 0)
        plsc.subcore_barrier()
        sl = pl.ds(s * ZERO_ROWS, ZERO_ROWS)

        @pl.when(c == 0)
        def _():
            pltpu.sync_copy(acc.at[sl], out0_h.at[sl])

        @pl.when(c == 1)
        def _():
            pltpu.sync_copy(acc.at[sl], out1_h.at[sl])

    return kdeg(dstp, ones_h, zeros_h)


def _conv_pass(ga, gb, srcp, dstp, zeros_h):
    @functools.partial(
        pl.kernel,
        out_type=(
            jax.ShapeDtypeStruct((ACC_ROWS, LANES), jnp.float32),
            jax.ShapeDtypeStruct((ACC_ROWS, LANES), jnp.float32),
        ),
        mesh=_sc_mesh(),
        compiler_params=pltpu.CompilerParams(use_tc_tiling_on_sc=False),
        scratch_types=[
            pltpu.VMEM((K_PER_CHUNK, IDX_W), jnp.int32),
            pltpu.VMEM((K_PER_CHUNK, IDX_W), jnp.int32),
            pltpu.VMEM((CHUNK, LANES), jnp.float32),
            pltpu.VMEM_SHARED((ACC_ROWS, LANES), jnp.float32),
            pltpu.SemaphoreType.DMA,
        ],
    )
    def kconv(ga_h, gb_h, srcp_h, dstp_h, zeros_hbm, outa_h, outb_h,
              sidx, didx, rows, acc, sem):
        c = lax.axis_index("c")
        s = lax.axis_index("s")
        pltpu.sync_copy(zeros_hbm, acc.at[pl.ds(s * ZERO_ROWS, ZERO_ROWS)])
        plsc.subcore_barrier()
        base = s * (CONV_CHUNKS * K_PER_CHUNK)

        def body(i, carry):
            rb = base + i * K_PER_CHUNK
            pltpu.sync_copy(srcp_h.at[pl.ds(rb, K_PER_CHUNK)], sidx)
            pltpu.sync_copy(dstp_h.at[pl.ds(rb, K_PER_CHUNK)], didx)

            @pl.when(c == 0)
            def _():
                descs = [
                    pltpu.async_copy(
                        ga_h.at[sidx.at[j]],
                        rows.at[pl.ds(j * IDX_W, IDX_W)], sem)
                    for j in range(K_PER_CHUNK)
                ]
                for d in descs:
                    d.wait()

            @pl.when(c == 1)
            def _():
                descs = [
                    pltpu.async_copy(
                        gb_h.at[sidx.at[j]],
                        rows.at[pl.ds(j * IDX_W, IDX_W)], sem)
                    for j in range(K_PER_CHUNK)
                ]
                for d in descs:
                    d.wait()

            for j in range(K_PER_CHUNK):
                pltpu.sync_copy(rows.at[pl.ds(j * IDX_W, IDX_W)],
                                acc.at[didx.at[j]], add=True)
            return carry

        lax.fori_loop(0, CONV_CHUNKS, body, 0)
        plsc.subcore_barrier()
        sl = pl.ds(s * ZERO_ROWS, ZERO_ROWS)

        @pl.when(c == 0)
        def _():
            pltpu.sync_copy(acc.at[sl], outa_h.at[sl])

        @pl.when(c == 1)
        def _():
            pltpu.sync_copy(acc.at[sl], outb_h.at[sl])

    return kconv(ga, gb, srcp, dstp, zeros_h)


def _dinv_block(d0, d1):
    return lax.rsqrt(d0 + d1 + 1.0)


def _tc1(x, W1, deg0, deg1):
    def body(x_r, w_r, d0_r, d1_r, ga_r, gb_r):
        dinv = _dinv_block(d0_r[...], d1_r[...])
        h = jnp.dot(x_r[...], w_r[...], preferred_element_type=jnp.float32)
        ga_r[...] = h[:, :LANES] * dinv
        gb_r[...] = h[:, LANES:] * dinv

    return pl.pallas_call(
        body,
        grid=(GRID,),
        in_specs=[
            pl.BlockSpec((BLK, x.shape[1]), lambda i: (i, 0)),
            pl.BlockSpec(W1.shape, lambda i: (0, 0)),
            pl.BlockSpec((BLK, LANES), lambda i: (i, 0)),
            pl.BlockSpec((BLK, LANES), lambda i: (i, 0)),
        ],
        out_specs=(
            pl.BlockSpec((BLK, LANES), lambda i: (i, 0)),
            pl.BlockSpec((BLK, LANES), lambda i: (i, 0)),
        ),
        out_shape=(
            jax.ShapeDtypeStruct((N_NODES, LANES), jnp.float32),
            jax.ShapeDtypeStruct((N_NODES, LANES), jnp.float32),
        ),
    )(x, W1, deg0, deg1)


def _tc2(deg0, deg1, acc_a, acc_b, ga, gb, b1, W2):
    def body(d0_r, d1_r, aa_r, ab_r, ga_r, gb_r, b1_r, w2_r, oa_r, ob_r):
        dinv = _dinv_block(d0_r[...], d1_r[...])
        ha = (aa_r[...] + ga_r[...]) * dinv
        hb = (ab_r[...] + gb_r[...]) * dinv
        h1 = jnp.concatenate([ha, hb], axis=1) + b1_r[...]
        h1 = jnp.maximum(h1, 0.0)
        h2 = jnp.dot(h1, w2_r[...], preferred_element_type=jnp.float32)
        oa_r[...] = h2[:, :LANES] * dinv
        ob_r[...] = h2[:, LANES:] * dinv

    blk16 = pl.BlockSpec((BLK, LANES), lambda i: (i, 0))
    return pl.pallas_call(
        body,
        grid=(GRID,),
        in_specs=[
            blk16, blk16, blk16, blk16, blk16, blk16,
            pl.BlockSpec(b1.shape, lambda i: (0, 0)),
            pl.BlockSpec(W2.shape, lambda i: (0, 0)),
        ],
        out_specs=(blk16, blk16),
        out_shape=(
            jax.ShapeDtypeStruct((N_NODES, LANES), jnp.float32),
            jax.ShapeDtypeStruct((N_NODES, LANES), jnp.float32),
        ),
    )(deg0, deg1, acc_a, acc_b, ga, gb, b1, W2)


def _tc3(deg0, deg1, acc_a, acc_b, ga, gb, b2, x, WL1x, WL1h, bL1, WL2, bL2):
    def body(d0_r, d1_r, aa_r, ab_r, ga_r, gb_r, b2_r, x_r,
             wx_r, wh_r, bl1_r, wl2_r, bl2_r, o_r):
        dinv = _dinv_block(d0_r[...], d1_r[...])
        ha = (aa_r[...] + ga_r[...]) * dinv
        hb = (ab_r[...] + gb_r[...]) * dinv
        h = jnp.concatenate([ha, hb], axis=1) + b2_r[...]
        h = jnp.maximum(h, 0.0)
        z = (jnp.dot(x_r[...], wx_r[...], preferred_element_type=jnp.float32)
             + jnp.dot(h, wh_r[...], preferred_element_type=jnp.float32)
             + bl1_r[...])
        z = jnp.maximum(z, 0.0)
        o = jnp.dot(z, wl2_r[...], preferred_element_type=jnp.float32)
        o_r[...] = jax.nn.sigmoid(o + bl2_r[...])

    blk16 = pl.BlockSpec((BLK, LANES), lambda i: (i, 0))
    full = lambda a: pl.BlockSpec(a.shape, lambda i: (0, 0))
    return pl.pallas_call(
        body,
        grid=(GRID,),
        in_specs=[
            blk16, blk16, blk16, blk16, blk16, blk16,
            full(b2),
            pl.BlockSpec((BLK, x.shape[1]), lambda i: (i, 0)),
            full(WL1x), full(WL1h), full(bL1), full(WL2), full(bL2),
        ],
        out_specs=pl.BlockSpec((BLK, 1), lambda i: (i, 0)),
        out_shape=jax.ShapeDtypeStruct((N_NODES, 1), jnp.float32),
    )(deg0, deg1, acc_a, acc_b, ga, gb, b2, x, WL1x, WL1h, bL1, WL2, bL2)


def kernel(x, edge_index, W1, b1, W2, b2, WL1, bL1, WL2, bL2):
    n_nodes = x.shape[0]
    n_edges = edge_index.shape[1]
    assert n_nodes == N_NODES and n_edges == N_EDGES

    src = edge_index[0].astype(jnp.int32)
    dst = edge_index[1].astype(jnp.int32)
    pad = EDGES_PAD - N_EDGES
    srcp = jnp.concatenate([src, jnp.zeros((pad,), jnp.int32)])
    dstp = jnp.concatenate([dst, jnp.full((pad,), N_NODES, jnp.int32)])
    srcp = srcp.reshape(IDX_ROWS, IDX_W)
    dstp = dstp.reshape(IDX_ROWS, IDX_W)

    ones_h = jnp.ones((IDX_W, LANES), jnp.float32)
    zeros_h = jnp.zeros((ZERO_ROWS, LANES), jnp.float32)

    deg0, deg1 = _deg_pass(dstp, ones_h, zeros_h)

    g1a, g1b = _tc1(x, W1, deg0, deg1)
    acc1a, acc1b = _conv_pass(g1a, g1b, srcp, dstp, zeros_h)

    b1r = b1.reshape(1, -1)
    g2a, g2b = _tc2(deg0, deg1, acc1a, acc1b, g1a, g1b, b1r, W2)
    acc2a, acc2b = _conv_pass(g2a, g2b, srcp, dstp, zeros_h)

    b2r = b2.reshape(1, -1)
    in_ch = x.shape[1]
    WL1x = WL1[:in_ch]
    WL1h = WL1[in_ch:]
    bL1r = bL1.reshape(1, -1)
    bL2r = bL2.reshape(1, -1)
    return _tc3(deg0, deg1, acc2a, acc2b, g2a, g2b, b2r, x,
                WL1x, WL1h, bL1r, WL2, bL2r)


# speedup vs baseline: 29.9394x; 1.3300x over previous
import functools

import jax
import jax.numpy as jnp
from jax import lax
from jax.experimental import pallas as pl
from jax.experimental.pallas import tpu as pltpu
from jax.experimental.pallas import tpu_sc as plsc

N_NODES = 100000
N_EDGES = 1600000
LANES = 16
IDX_W = 128
K_PER_CHUNK = 8
CHUNK = IDX_W * K_PER_CHUNK
EDGES_PAD = 98 * 16 * CHUNK
IDX_ROWS = EDGES_PAD // IDX_W
CONV_CHUNKS = EDGES_PAD // (16 * CHUNK)
DEG_CHUNKS = EDGES_PAD // (32 * CHUNK)
NP = 100096
RP = NP // 8
ZERO_ROWS = NP // 16
BLKP = 736
BLKP3 = 184


def _sc_mesh():
    return plsc.VectorSubcoreMesh(core_axis_name="c", subcore_axis_name="s")


def _fill_rows(ref, n_rows, val):
    v = jnp.full((LANES,), val, jnp.float32)

    def body(i, carry):
        ref[i] = v
        return carry

    lax.fori_loop(0, n_rows, body, 0)


def _zero_acc(acc, zbuf, s):
    base = s * ZERO_ROWS
    nz = zbuf.shape[0]
    done = 0
    while done < ZERO_ROWS:
        step = min(nz, ZERO_ROWS - done)
        pltpu.sync_copy(zbuf.at[pl.ds(0, step)],
                        acc.at[pl.ds(base + done, step)])
        done += step


def _deg_pass(ei):
    @functools.partial(
        pl.kernel,
        out_type=jax.ShapeDtypeStruct((2, NP, LANES), jnp.float32),
        mesh=_sc_mesh(),
        compiler_params=pltpu.CompilerParams(use_tc_tiling_on_sc=False),
        scratch_types=[
            pltpu.VMEM((K_PER_CHUNK, IDX_W), jnp.int32),
            pltpu.VMEM((IDX_W, LANES), jnp.float32),
            pltpu.VMEM((1024, LANES), jnp.float32),
            pltpu.VMEM_SHARED((NP, LANES), jnp.float32),
        ],
    )
    def kdeg(ei_h, out_h, didx, ones_v, zbuf, acc):
        c = lax.axis_index("c")
        s = lax.axis_index("s")
        _fill_rows(ones_v, IDX_W, 1.0)
        _fill_rows(zbuf, 1024, 0.0)
        _zero_acc(acc, zbuf, s)
        plsc.subcore_barrier()
        wid = s * 2 + c
        base = wid * (DEG_CHUNKS * K_PER_CHUNK)

        def body(i, carry):
            rb = base + i * K_PER_CHUNK
            pltpu.sync_copy(ei_h.at[1, pl.ds(rb, K_PER_CHUNK)], didx)
            for j in range(K_PER_CHUNK):
                pltpu.sync_copy(ones_v, acc.at[didx.at[j]], add=True)
            return carry

        lax.fori_loop(0, DEG_CHUNKS, body, 0)
        plsc.subcore_barrier()
        sl = pl.ds(s * ZERO_ROWS, ZERO_ROWS)

        @pl.when(c == 0)
        def _():
            pltpu.sync_copy(acc.at[sl], out_h.at[0, sl])

        @pl.when(c == 1)
        def _():
            pltpu.sync_copy(acc.at[sl], out_h.at[1, sl])

    return kdeg(ei)


def _conv_pass(g, ei):
    @functools.partial(
        pl.kernel,
        out_type=jax.ShapeDtypeStruct((2, NP, LANES), jnp.float32),
        mesh=_sc_mesh(),
        compiler_params=pltpu.CompilerParams(use_tc_tiling_on_sc=False),
        scratch_types=[
            pltpu.VMEM((K_PER_CHUNK, IDX_W), jnp.int32),
            pltpu.VMEM((K_PER_CHUNK, IDX_W), jnp.int32),
            pltpu.VMEM((CHUNK, LANES), jnp.float32),
            pltpu.VMEM_SHARED((NP, LANES), jnp.float32),
            pltpu.SemaphoreType.DMA,
        ],
    )
    def kconv(g_h, ei_h, out_h, sidx, didx, rows, acc, sem):
        c = lax.axis_index("c")
        s = lax.axis_index("s")
        _fill_rows(rows, CHUNK, 0.0)
        _zero_acc(acc, rows, s)
        plsc.subcore_barrier()
        base = s * (CONV_CHUNKS * K_PER_CHUNK)

        def body(i, carry):
            rb = base + i * K_PER_CHUNK
            pltpu.sync_copy(ei_h.at[0, pl.ds(rb, K_PER_CHUNK)], sidx)
            pltpu.sync_copy(ei_h.at[1, pl.ds(rb, K_PER_CHUNK)], didx)

            @pl.when(c == 0)
            def _():
                descs = [
                    pltpu.async_copy(
                        g_h.at[0].at[sidx.at[j]],
                        rows.at[pl.ds(j * IDX_W, IDX_W)], sem)
                    for j in range(K_PER_CHUNK)
                ]
                for d in descs:
                    d.wait()

            @pl.when(c == 1)
            def _():
                descs = [
                    pltpu.async_copy(
                        g_h.at[1].at[sidx.at[j]],
                        rows.at[pl.ds(j * IDX_W, IDX_W)], sem)
                    for j in range(K_PER_CHUNK)
                ]
                for d in descs:
                    d.wait()

            for j in range(K_PER_CHUNK):
                pltpu.sync_copy(rows.at[pl.ds(j * IDX_W, IDX_W)],
                                acc.at[didx.at[j]], add=True)
            return carry

        lax.fori_loop(0, CONV_CHUNKS, body, 0)
        plsc.subcore_barrier()
        sl = pl.ds(s * ZERO_ROWS, ZERO_ROWS)

        @pl.when(c == 0)
        def _():
            pltpu.sync_copy(acc.at[sl], out_h.at[0, sl])

        @pl.when(c == 1)
        def _():
            pltpu.sync_copy(acc.at[sl], out_h.at[1, sl])

    return kconv(g, ei)


def _tc1(xp, W1bd, degp):
    def body(x_r, w_r, d_r, o_r):
        d = d_r[...]
        dinv = lax.rsqrt(d[0] + d[1] + 1.0)
        h = jnp.dot(x_r[...], w_r[...], preferred_element_type=jnp.float32)
        o_r[0] = h[:, :128] * dinv
        o_r[1] = h[:, 128:] * dinv

    return pl.pallas_call(
        body,
        grid=(RP // BLKP,),
        in_specs=[
            pl.BlockSpec((BLKP, 128), lambda i: (i, 0)),
            pl.BlockSpec(W1bd.shape, lambda i: (0, 0)),
            pl.BlockSpec((2, BLKP, 128), lambda i: (0, i, 0)),
        ],
        out_specs=pl.BlockSpec((2, BLKP, 128), lambda i: (0, i, 0)),
        out_shape=jax.ShapeDtypeStruct((2, RP, 128), jnp.float32),
    )(xp, W1bd, degp)


def _tc2(degp, acc1, g1, b1p, W2bd):
    def body(d_r, a_r, g_r, b_r, w_r, o_r):
        d = d_r[...]
        dinv = lax.rsqrt(d[0] + d[1] + 1.0)
        b = b_r[...]
        ra = jnp.maximum((a_r[0] + g_r[0]) * dinv + b[:, :128], 0.0)
        rb = jnp.maximum((a_r[1] + g_r[1]) * dinv + b[:, 128:], 0.0)
        h = jnp.dot(jnp.concatenate([ra, rb], axis=1), w_r[...],
                    preferred_element_type=jnp.float32)
        o_r[0] = h[:, :128] * dinv
        o_r[1] = h[:, 128:] * dinv

    blk2 = pl.BlockSpec((2, BLKP, 128), lambda i: (0, i, 0))
    return pl.pallas_call(
        body,
        grid=(RP // BLKP,),
        in_specs=[
            blk2, blk2, blk2,
            pl.BlockSpec(b1p.shape, lambda i: (0, 0)),
            pl.BlockSpec(W2bd.shape, lambda i: (0, 0)),
        ],
        out_specs=blk2,
        out_shape=jax.ShapeDtypeStruct((2, RP, 128), jnp.float32),
    )(degp, acc1, g1, b1p, W2bd)


def _tc3(degp, acc2, g2, b2p, xp, WL1x, WL1a, WL1b, bL1, WL2, bL2):
    def body(d_r, a_r, g_r, b_r, x_r, wx_r, wa_r, wb_r, bl1_r, wl2_r,
             bl2_r, o_r):
        d = d_r[...]
        dinv = lax.rsqrt(d[0] + d[1] + 1.0)
        b = b_r[...]
        ha = jnp.maximum((a_r[0] + g_r[0]) * dinv + b[:, :128], 0.0)
        hb = jnp.maximum((a_r[1] + g_r[1]) * dinv + b[:, 128:], 0.0)
        xt = x_r[...]
        for j in range(8):
            sl = slice(LANES * j, LANES * (j + 1))
            z = (jnp.dot(xt[:, sl], wx_r[...],
                         preferred_element_type=jnp.float32)
                 + jnp.dot(ha[:, sl], wa_r[...],
                           preferred_element_type=jnp.float32)
                 + jnp.dot(hb[:, sl], wb_r[...],
                           preferred_element_type=jnp.float32)
                 + bl1_r[...])
            z = jnp.maximum(z, 0.0)
            o = jnp.dot(z, wl2_r[...], preferred_element_type=jnp.float32)
            o_r[:, pl.ds(j, 1)] = jax.nn.sigmoid(o + bl2_r[...])

    blk2 = pl.BlockSpec((2, BLKP3, 128), lambda i: (0, i, 0))
    full = lambda a: pl.BlockSpec(a.shape, lambda i: (0, 0))
    return pl.pallas_call(
        body,
        grid=(RP // BLKP3,),
        in_specs=[
            blk2, blk2, blk2,
            pl.BlockSpec(b2p.shape, lambda i: (0, 0)),
            pl.BlockSpec((BLKP3, 128), lambda i: (i, 0)),
            full(WL1x), full(WL1a), full(WL1b), full(bL1), full(WL2),
            full(bL2),
        ],
        out_specs=pl.BlockSpec((BLKP3, 8), lambda i: (i, 0)),
        out_shape=jax.ShapeDtypeStruct((RP, 8), jnp.float32),
    )(degp, acc2, g2, b2p, xp, WL1x, WL1a, WL1b, bL1, WL2, bL2)


def _block_diag8(w):
    k = w.shape[1]
    out = jnp.zeros((128, 8 * k), jnp.float32)
    for j in range(8):
        out = out.at[16 * j:16 * (j + 1), k * j:k * (j + 1)].set(w)
    return out


def kernel(x, edge_index, W1, b1, W2, b2, WL1, bL1, WL2, bL2):
    n_nodes = x.shape[0]
    in_ch = x.shape[1]
    assert n_nodes == N_NODES and edge_index.shape[1] == N_EDGES

    pad = EDGES_PAD - N_EDGES
    pad_src = (jnp.arange(pad, dtype=jnp.int32) * 17) % N_NODES
    pad_dst = N_NODES + (jnp.arange(pad, dtype=jnp.int32) % (NP - N_NODES))
    ei = jnp.concatenate(
        [edge_index.astype(jnp.int32),
         jnp.stack([pad_src, pad_dst])], axis=1).reshape(2, IDX_ROWS, IDX_W)

    xpad = jnp.zeros((NP, LANES), jnp.float32).at[:N_NODES, :in_ch].set(x)
    xp = xpad.reshape(RP, 128)

    W1p = jnp.zeros((LANES, 32), jnp.float32).at[:in_ch].set(W1)
    W1bd = jnp.concatenate(
        [_block_diag8(W1p[:, :16]), _block_diag8(W1p[:, 16:])], axis=1)
    W2bd = jnp.block(
        [[_block_diag8(W2[:16, :16]), _block_diag8(W2[:16, 16:])],
         [_block_diag8(W2[16:, :16]), _block_diag8(W2[16:, 16:])]])
    b1p = jnp.concatenate([jnp.tile(b1[:16], 8), jnp.tile(b1[16:], 8)])
    b1p = b1p.reshape(1, 256)
    b2p = jnp.concatenate([jnp.tile(b2[:16], 8), jnp.tile(b2[16:], 8)])
    b2p = b2p.reshape(1, 256)
    WL1x = jnp.zeros((LANES, 1024), jnp.float32).at[:in_ch].set(WL1[:in_ch])
    WL1a = WL1[in_ch:in_ch + 16]
    WL1b = WL1[in_ch + 16:in_ch + 32]
    bL1r = bL1.reshape(1, -1)
    bL2r = bL2.reshape(1, -1)

    degp = _deg_pass(ei).reshape(2, RP, 128)

    g1 = _tc1(xp, W1bd, degp)
    acc1 = _conv_pass(g1.reshape(2, NP, LANES), ei).reshape(2, RP, 128)

    g2 = _tc2(degp, acc1, g1, b1p, W2bd)
    acc2 = _conv_pass(g2.reshape(2, NP, LANES), ei).reshape(2, RP, 128)

    out = _tc3(degp, acc2, g2, b2p, xp, WL1x, WL1a, WL1b, bL1r, WL2, bL2r)
    return out.reshape(NP, 1)[:N_NODES]


# speedup vs baseline: 32.5445x; 1.0870x over previous
import functools

import jax
import jax.numpy as jnp
from jax import lax
from jax.experimental import pallas as pl
from jax.experimental.pallas import tpu as pltpu
from jax.experimental.pallas import tpu_sc as plsc

N_NODES = 100000
N_EDGES = 1600000
LANES = 16
IDX_W = 128
K_PER_CHUNK = 6
CHUNK = IDX_W * K_PER_CHUNK
EDGES_PAD = 132 * 16 * CHUNK
IDX_ROWS = EDGES_PAD // IDX_W
CONV_CHUNKS = EDGES_PAD // (16 * CHUNK)
DEG_CHUNKS = EDGES_PAD // (32 * CHUNK)
NBUF = 2
ZB = 136
NP = 100096
RP = NP // 8
ZERO_ROWS = NP // 16
BLKP = 736
BLKP3 = 184


def _sc_mesh():
    return plsc.VectorSubcoreMesh(core_axis_name="c", subcore_axis_name="s")


def _fill_rows(ref, n_rows, val):
    v = jnp.full((LANES,), val, jnp.float32)

    def body(i, carry):
        ref[i] = v
        return carry

    lax.fori_loop(0, n_rows, body, 0)


def _zero_acc(acc, zbuf, s):
    base = s * ZERO_ROWS
    nz = zbuf.shape[0]
    done = 0
    while done < ZERO_ROWS:
        step = min(nz, ZERO_ROWS - done)
        pltpu.sync_copy(zbuf.at[pl.ds(0, step)],
                        acc.at[pl.ds(base + done, step)])
        done += step


def _deg_pass(ei):
    @functools.partial(
        pl.kernel,
        out_type=jax.ShapeDtypeStruct((2, NP, LANES), jnp.float32),
        mesh=_sc_mesh(),
        compiler_params=pltpu.CompilerParams(use_tc_tiling_on_sc=False),
        scratch_types=[
            pltpu.VMEM((2, K_PER_CHUNK, IDX_W), jnp.int32),
            pltpu.VMEM((IDX_W, LANES), jnp.float32),
            pltpu.VMEM((ZB, LANES), jnp.float32),
            pltpu.VMEM_SHARED((NP, LANES), jnp.float32),
            pltpu.SemaphoreType.DMA,
            pltpu.SemaphoreType.DMA,
        ],
    )
    def kdeg(ei_h, out_h, didx, ones_v, zbuf, acc, sem0, sem1):
        c = lax.axis_index("c")
        s = lax.axis_index("s")
        sems = [sem0, sem1]
        _fill_rows(ones_v, IDX_W, 1.0)
        _fill_rows(zbuf, ZB, 0.0)
        _zero_acc(acc, zbuf, s)
        plsc.subcore_barrier()
        wid = s * 2 + c
        base = wid * (DEG_CHUNKS * K_PER_CHUNK)

        def stage(b, i):
            rb = base + i * K_PER_CHUNK
            pltpu.sync_copy(ei_h.at[1, pl.ds(rb, K_PER_CHUNK)], didx.at[b])

        def fire(b):
            for j in range(K_PER_CHUNK):
                pltpu.async_copy(ones_v, acc.at[didx.at[b, j]], sems[b],
                                 add=True)

        def drain(b):
            pltpu.make_async_copy(
                ones_v, acc.at[didx.at[b, 0]], sems[b]).wait()

        stage(0, 0)

        def body(t, carry):
            for k in range(2):
                i = 2 * t + k
                fire(k)

                @pl.when(i >= 1)
                def _():
                    for _j in range(K_PER_CHUNK):
                        drain(1 - k)

                @pl.when(i + 1 < DEG_CHUNKS)
                def _():
                    stage(1 - k, i + 1)
            return carry

        lax.fori_loop(0, DEG_CHUNKS // 2, body, 0)
        for _j in range(K_PER_CHUNK):
            drain(1)
        plsc.subcore_barrier()
        sl = pl.ds(s * ZERO_ROWS, ZERO_ROWS)

        @pl.when(c == 0)
        def _():
            pltpu.sync_copy(acc.at[sl], out_h.at[0, sl])

        @pl.when(c == 1)
        def _():
            pltpu.sync_copy(acc.at[sl], out_h.at[1, sl])

    return kdeg(ei)


def _conv_pass(g, ei):
    @functools.partial(
        pl.kernel,
        out_type=jax.ShapeDtypeStruct((2, NP, LANES), jnp.float32),
        mesh=_sc_mesh(),
        compiler_params=pltpu.CompilerParams(use_tc_tiling_on_sc=False),
        scratch_types=[
            pltpu.VMEM((NBUF, K_PER_CHUNK, IDX_W), jnp.int32),
            pltpu.VMEM((NBUF, K_PER_CHUNK, IDX_W), jnp.int32),
            pltpu.VMEM((NBUF, CHUNK, LANES), jnp.float32),
            pltpu.VMEM((ZB, LANES), jnp.float32),
            pltpu.VMEM_SHARED((NP, LANES), jnp.float32),
        ] + [pltpu.SemaphoreType.DMA] * 4,
    )
    def kconv(g_h, ei_h, out_h, sidx, didx, rows, zbuf, acc,
              sg0, sg1, ss0, ss1):
        c = lax.axis_index("c")
        s = lax.axis_index("s")
        semg = [sg0, sg1]
        sems = [ss0, ss1]
        _fill_rows(zbuf, ZB, 0.0)
        _zero_acc(acc, zbuf, s)
        plsc.subcore_barrier()
        base = s * (CONV_CHUNKS * K_PER_CHUNK)

        def stage(b, i):
            rb = base + i * K_PER_CHUNK
            pltpu.sync_copy(ei_h.at[0, pl.ds(rb, K_PER_CHUNK)], sidx.at[b])
            pltpu.sync_copy(ei_h.at[1, pl.ds(rb, K_PER_CHUNK)], didx.at[b])

        def fire_g(b):
            @pl.when(c == 0)
            def _():
                for j in range(K_PER_CHUNK):
                    pltpu.async_copy(g_h.at[0].at[sidx.at[b, j]],
                                     rows.at[b, pl.ds(j * IDX_W, IDX_W)],
                                     semg[b])

            @pl.when(c == 1)
            def _():
                for j in range(K_PER_CHUNK):
                    pltpu.async_copy(g_h.at[1].at[sidx.at[b, j]],
                                     rows.at[b, pl.ds(j * IDX_W, IDX_W)],
                                     semg[b])

        def wait_g(b):
            pltpu.make_async_copy(
                g_h.at[0].at[sidx.at[b, 0]], rows.at[b], semg[b]).wait()

        def fire_s(b):
            for j in range(K_PER_CHUNK):
                pltpu.async_copy(rows.at[b, pl.ds(j * IDX_W, IDX_W)],
                                 acc.at[didx.at[b, j]], sems[b], add=True)

        def wait_s(b):
            pltpu.make_async_copy(
                rows.at[b], acc.at[didx.at[b, 0]], sems[b]).wait()

        stage(0, 0)
        fire_g(0)

        def body(t, carry):
            for k in range(2):
                i = 2 * t + k
                wait_g(k)
                fire_s(k)

                @pl.when(i >= 1)
                def _():
                    wait_s(1 - k)

                @pl.when(i + 1 < CONV_CHUNKS)
                def _():
                    stage(1 - k, i + 1)
                    fire_g(1 - k)
            return carry

        lax.fori_loop(0, CONV_CHUNKS // 2, body, 0)
        wait_s(1)
        plsc.subcore_barrier()
        sl = pl.ds(s * ZERO_ROWS, ZERO_ROWS)

        @pl.when(c == 0)
        def _():
            pltpu.sync_copy(acc.at[sl], out_h.at[0, sl])

        @pl.when(c == 1)
        def _():
            pltpu.sync_copy(acc.at[sl], out_h.at[1, sl])

    return kconv(g, ei)


def _tc1(xp, W1bd, degp):
    def body(x_r, w_r, d_r, o_r):
        d = d_r[...]
        dinv = lax.rsqrt(d[0] + d[1] + 1.0)
        h = jnp.dot(x_r[...], w_r[...], preferred_element_type=jnp.float32)
        o_r[0] = h[:, :128] * dinv
        o_r[1] = h[:, 128:] * dinv

    return pl.pallas_call(
        body,
        grid=(RP // BLKP,),
        in_specs=[
            pl.BlockSpec((BLKP, 128), lambda i: (i, 0)),
            pl.BlockSpec(W1bd.shape, lambda i: (0, 0)),
            pl.BlockSpec((2, BLKP, 128), lambda i: (0, i, 0)),
        ],
        out_specs=pl.BlockSpec((2, BLKP, 128), lambda i: (0, i, 0)),
        out_shape=jax.ShapeDtypeStruct((2, RP, 128), jnp.float32),
    )(xp, W1bd, degp)


def _tc2(degp, acc1, g1, b1p, W2bd):
    def body(d_r, a_r, g_r, b_r, w_r, o_r):
        d = d_r[...]
        dinv = lax.rsqrt(d[0] + d[1] + 1.0)
        b = b_r[...]
        ra = jnp.maximum((a_r[0] + g_r[0]) * dinv + b[:, :128], 0.0)
        rb = jnp.maximum((a_r[1] + g_r[1]) * dinv + b[:, 128:], 0.0)
        h = jnp.dot(jnp.concatenate([ra, rb], axis=1), w_r[...],
                    preferred_element_type=jnp.float32)
        o_r[0] = h[:, :128] * dinv
        o_r[1] = h[:, 128:] * dinv

    blk2 = pl.BlockSpec((2, BLKP, 128), lambda i: (0, i, 0))
    return pl.pallas_call(
        body,
        grid=(RP // BLKP,),
        in_specs=[
            blk2, blk2, blk2,
            pl.BlockSpec(b1p.shape, lambda i: (0, 0)),
            pl.BlockSpec(W2bd.shape, lambda i: (0, 0)),
        ],
        out_specs=blk2,
        out_shape=jax.ShapeDtypeStruct((2, RP, 128), jnp.float32),
    )(degp, acc1, g1, b1p, W2bd)


def _tc3(degp, acc2, g2, b2p, xp, WL1x, WL1a, WL1b, bL1, WL2, bL2):
    def body(d_r, a_r, g_r, b_r, x_r, wx_r, wa_r, wb_r, bl1_r, wl2_r,
             bl2_r, o_r):
        d = d_r[...]
        dinv = lax.rsqrt(d[0] + d[1] + 1.0)
        b = b_r[...]
        ha = jnp.maximum((a_r[0] + g_r[0]) * dinv + b[:, :128], 0.0)
        hb = jnp.maximum((a_r[1] + g_r[1]) * dinv + b[:, 128:], 0.0)
        xt = x_r[...]
        for j in range(8):
            sl = slice(LANES * j, LANES * (j + 1))
            z = (jnp.dot(xt[:, sl], wx_r[...],
                         preferred_element_type=jnp.float32)
                 + jnp.dot(ha[:, sl], wa_r[...],
                           preferred_element_type=jnp.float32)
                 + jnp.dot(hb[:, sl], wb_r[...],
                           preferred_element_type=jnp.float32)
                 + bl1_r[...])
            z = jnp.maximum(z, 0.0)
            o = jnp.dot(z, wl2_r[...], preferred_element_type=jnp.float32)
            o_r[:, pl.ds(j, 1)] = jax.nn.sigmoid(o + bl2_r[...])

    blk2 = pl.BlockSpec((2, BLKP3, 128), lambda i: (0, i, 0))
    full = lambda a: pl.BlockSpec(a.shape, lambda i: (0, 0))
    return pl.pallas_call(
        body,
        grid=(RP // BLKP3,),
        in_specs=[
            blk2, blk2, blk2,
            pl.BlockSpec(b2p.shape, lambda i: (0, 0)),
            pl.BlockSpec((BLKP3, 128), lambda i: (i, 0)),
            full(WL1x), full(WL1a), full(WL1b), full(bL1), full(WL2),
            full(bL2),
        ],
        out_specs=pl.BlockSpec((BLKP3, 8), lambda i: (i, 0)),
        out_shape=jax.ShapeDtypeStruct((RP, 8), jnp.float32),
    )(degp, acc2, g2, b2p, xp, WL1x, WL1a, WL1b, bL1, WL2, bL2)


def _block_diag8(w):
    k = w.shape[1]
    out = jnp.zeros((128, 8 * k), jnp.float32)
    for j in range(8):
        out = out.at[16 * j:16 * (j + 1), k * j:k * (j + 1)].set(w)
    return out


def kernel(x, edge_index, W1, b1, W2, b2, WL1, bL1, WL2, bL2):
    n_nodes = x.shape[0]
    in_ch = x.shape[1]
    assert n_nodes == N_NODES and edge_index.shape[1] == N_EDGES

    pad = EDGES_PAD - N_EDGES
    pad_src = (jnp.arange(pad, dtype=jnp.int32) * 17) % N_NODES
    pad_dst = N_NODES + (jnp.arange(pad, dtype=jnp.int32) % (NP - N_NODES))
    ei = jnp.concatenate(
        [edge_index.astype(jnp.int32),
         jnp.stack([pad_src, pad_dst])], axis=1).reshape(2, IDX_ROWS, IDX_W)

    xpad = jnp.zeros((NP, LANES), jnp.float32).at[:N_NODES, :in_ch].set(x)
    xp = xpad.reshape(RP, 128)

    W1p = jnp.zeros((LANES, 32), jnp.float32).at[:in_ch].set(W1)
    W1bd = jnp.concatenate(
        [_block_diag8(W1p[:, :16]), _block_diag8(W1p[:, 16:])], axis=1)
    W2bd = jnp.block(
        [[_block_diag8(W2[:16, :16]), _block_diag8(W2[:16, 16:])],
         [_block_diag8(W2[16:, :16]), _block_diag8(W2[16:, 16:])]])
    b1p = jnp.concatenate([jnp.tile(b1[:16], 8), jnp.tile(b1[16:], 8)])
    b1p = b1p.reshape(1, 256)
    b2p = jnp.concatenate([jnp.tile(b2[:16], 8), jnp.tile(b2[16:], 8)])
    b2p = b2p.reshape(1, 256)
    WL1x = jnp.zeros((LANES, 1024), jnp.float32).at[:in_ch].set(WL1[:in_ch])
    WL1a = WL1[in_ch:in_ch + 16]
    WL1b = WL1[in_ch + 16:in_ch + 32]
    bL1r = bL1.reshape(1, -1)
    bL2r = bL2.reshape(1, -1)

    degp = _deg_pass(ei).reshape(2, RP, 128)

    g1 = _tc1(xp, W1bd, degp)
    acc1 = _conv_pass(g1.reshape(2, NP, LANES), ei).reshape(2, RP, 128)

    g2 = _tc2(degp, acc1, g1, b1p, W2bd)
    acc2 = _conv_pass(g2.reshape(2, NP, LANES), ei).reshape(2, RP, 128)

    out = _tc3(degp, acc2, g2, b2p, xp, WL1x, WL1a, WL1b, bL1r, WL2, bL2r)
    return out.reshape(NP, 1)[:N_NODES]


# speedup vs baseline: 38.9435x; 1.1966x over previous
import functools

import jax
import jax.numpy as jnp
from jax import lax
from jax.experimental import pallas as pl
from jax.experimental.pallas import tpu as pltpu
from jax.experimental.pallas import tpu_sc as plsc

N_NODES = 100000
N_EDGES = 1600000
LANES = 16
IDX_W = 128
K_PER_CHUNK = 6
CHUNK = IDX_W * K_PER_CHUNK
EDGES_PAD = 132 * 16 * CHUNK
IDX_ROWS = EDGES_PAD // IDX_W
CONV_CHUNKS = EDGES_PAD // (16 * CHUNK)
DEG_CHUNKS = EDGES_PAD // (32 * CHUNK)
NBUF = 2
ZB = 136
NP = 100096
RP = NP // 8
ZERO_ROWS = NP // 16
BLKP = 736
BLKP3 = 184


def _sc_mesh():
    return plsc.VectorSubcoreMesh(core_axis_name="c", subcore_axis_name="s")


def _fill_rows(ref, n_rows, val):
    v = jnp.full((LANES,), val, jnp.float32)

    def body(i, carry):
        ref[i] = v
        return carry

    lax.fori_loop(0, n_rows, body, 0)


def _zero_acc(acc, zbuf, s):
    base = s * ZERO_ROWS
    nz = zbuf.shape[0]
    done = 0
    while done < ZERO_ROWS:
        step = min(nz, ZERO_ROWS - done)
        pltpu.sync_copy(zbuf.at[pl.ds(0, step)],
                        acc.at[pl.ds(base + done, step)])
        done += step


def _deg_pass(ei):
    @functools.partial(
        pl.kernel,
        out_type=jax.ShapeDtypeStruct((2, NP, LANES), jnp.float32),
        mesh=_sc_mesh(),
        compiler_params=pltpu.CompilerParams(use_tc_tiling_on_sc=False),
        scratch_types=[
            pltpu.VMEM((2, K_PER_CHUNK, IDX_W), jnp.int32),
            pltpu.VMEM((IDX_W, LANES), jnp.float32),
            pltpu.VMEM((ZB, LANES), jnp.float32),
            pltpu.VMEM_SHARED((NP, LANES), jnp.float32),
            pltpu.SemaphoreType.DMA,
            pltpu.SemaphoreType.DMA,
        ],
    )
    def kdeg(ei_h, out_h, didx, ones_v, zbuf, acc, sem0, sem1):
        c = lax.axis_index("c")
        s = lax.axis_index("s")
        sems = [sem0, sem1]
        _fill_rows(ones_v, IDX_W, 1.0)
        _fill_rows(zbuf, ZB, 0.0)
        _zero_acc(acc, zbuf, s)
        plsc.subcore_barrier()
        wid = s * 2 + c
        base = wid * (DEG_CHUNKS * K_PER_CHUNK)

        def stage(b, i):
            rb = base + i * K_PER_CHUNK
            pltpu.sync_copy(ei_h.at[1, pl.ds(rb, K_PER_CHUNK)], didx.at[b])

        def fire(b):
            for j in range(K_PER_CHUNK):
                pltpu.async_copy(ones_v, acc.at[didx.at[b, j]], sems[b],
                                 add=True)

        def drain(b):
            pltpu.make_async_copy(
                ones_v, acc.at[didx.at[b, 0]], sems[b]).wait()

        stage(0, 0)

        def body(t, carry):
            for k in range(2):
                i = 2 * t + k
                fire(k)

                @pl.when(i >= 1)
                def _():
                    for _j in range(K_PER_CHUNK):
                        drain(1 - k)

                @pl.when(i + 1 < DEG_CHUNKS)
                def _():
                    stage(1 - k, i + 1)
            return carry

        lax.fori_loop(0, DEG_CHUNKS // 2, body, 0)
        for _j in range(K_PER_CHUNK):
            drain(1)
        plsc.subcore_barrier()
        sl = pl.ds(s * ZERO_ROWS, ZERO_ROWS)

        @pl.when(c == 0)
        def _():
            pltpu.sync_copy(acc.at[sl], out_h.at[0, sl])

        @pl.when(c == 1)
        def _():
            pltpu.sync_copy(acc.at[sl], out_h.at[1, sl])

    return kdeg(ei)


def _conv_pass(g, ei):
    @functools.partial(
        pl.kernel,
        out_type=jax.ShapeDtypeStruct((2, NP, LANES), jnp.float32),
        mesh=_sc_mesh(),
        compiler_params=pltpu.CompilerParams(use_tc_tiling_on_sc=False),
        scratch_types=[
            pltpu.VMEM((NBUF, K_PER_CHUNK, IDX_W), jnp.int32),
            pltpu.VMEM((NBUF, K_PER_CHUNK, IDX_W), jnp.int32),
            pltpu.VMEM((NBUF, CHUNK, LANES), jnp.float32),
            pltpu.VMEM((ZB, LANES), jnp.float32),
            pltpu.VMEM_SHARED((NP, LANES), jnp.float32),
        ] + [pltpu.SemaphoreType.DMA] * 4,
    )
    def kconv(g_h, ei_h, out_h, sidx, didx, rows, zbuf, acc,
              sg0, sg1, ss0, ss1):
        c = lax.axis_index("c")
        s = lax.axis_index("s")
        semg = [sg0, sg1]
        sems = [ss0, ss1]
        _fill_rows(zbuf, ZB, 0.0)
        _zero_acc(acc, zbuf, s)
        plsc.subcore_barrier()
        base = s * (CONV_CHUNKS * K_PER_CHUNK)

        def stage(b, i):
            rb = base + i * K_PER_CHUNK
            pltpu.sync_copy(ei_h.at[0, pl.ds(rb, K_PER_CHUNK)], sidx.at[b])
            pltpu.sync_copy(ei_h.at[1, pl.ds(rb, K_PER_CHUNK)], didx.at[b])

        def fire_g(b):
            @pl.when(c == 0)
            def _():
                for j in range(K_PER_CHUNK):
                    pltpu.async_copy(g_h.at[0].at[sidx.at[b, j]],
                                     rows.at[b, pl.ds(j * IDX_W, IDX_W)],
                                     semg[b])

            @pl.when(c == 1)
            def _():
                for j in range(K_PER_CHUNK):
                    pltpu.async_copy(g_h.at[1].at[sidx.at[b, j]],
                                     rows.at[b, pl.ds(j * IDX_W, IDX_W)],
                                     semg[b])

        def wait_g(b):
            pltpu.make_async_copy(
                g_h.at[0].at[sidx.at[b, 0]], rows.at[b], semg[b]).wait()

        def fire_s(b):
            for j in range(K_PER_CHUNK):
                pltpu.async_copy(rows.at[b, pl.ds(j * IDX_W, IDX_W)],
                                 acc.at[didx.at[b, j]], sems[b], add=True)

        def wait_s(b):
            pltpu.make_async_copy(
                rows.at[b], acc.at[didx.at[b, 0]], sems[b]).wait()

        stage(0, 0)
        fire_g(0)

        def body(t, carry):
            for k in range(2):
                i = 2 * t + k
                wait_g(k)
                fire_s(k)

                @pl.when(i >= 1)
                def _():
                    wait_s(1 - k)

                @pl.when(i + 1 < CONV_CHUNKS)
                def _():
                    stage(1 - k, i + 1)
                    fire_g(1 - k)
            return carry

        lax.fori_loop(0, CONV_CHUNKS // 2, body, 0)
        wait_s(1)
        plsc.subcore_barrier()
        sl = pl.ds(s * ZERO_ROWS, ZERO_ROWS)

        @pl.when(c == 0)
        def _():
            pltpu.sync_copy(acc.at[sl], out_h.at[0, sl])

        @pl.when(c == 1)
        def _():
            pltpu.sync_copy(acc.at[sl], out_h.at[1, sl])

    return kconv(g, ei)


def _tc1(xp, W1bd, degp):
    def body(x_r, w_r, d_r, o_r):
        d = d_r[...]
        dinv = lax.rsqrt(d[0] + d[1] + 1.0)
        h = jnp.dot(x_r[...], w_r[...], preferred_element_type=jnp.float32)
        o_r[0] = h[:, :128] * dinv
        o_r[1] = h[:, 128:] * dinv

    return pl.pallas_call(
        body,
        grid=(RP // BLKP,),
        in_specs=[
            pl.BlockSpec((BLKP, 128), lambda i: (i, 0)),
            pl.BlockSpec(W1bd.shape, lambda i: (0, 0)),
            pl.BlockSpec((2, BLKP, 128), lambda i: (0, i, 0)),
        ],
        out_specs=pl.BlockSpec((2, BLKP, 128), lambda i: (0, i, 0)),
        out_shape=jax.ShapeDtypeStruct((2, RP, 128), jnp.float32),
    )(xp, W1bd, degp)


def _tc2(degp, acc1, g1, b1p, W2bd):
    def body(d_r, a_r, g_r, b_r, w_r, o_r):
        d = d_r[...]
        dinv = lax.rsqrt(d[0] + d[1] + 1.0)
        b = b_r[...]
        ra = jnp.maximum((a_r[0] + g_r[0]) * dinv + b[:, :128], 0.0)
        rb = jnp.maximum((a_r[1] + g_r[1]) * dinv + b[:, 128:], 0.0)
        h = jnp.dot(jnp.concatenate([ra, rb], axis=1), w_r[...],
                    preferred_element_type=jnp.float32)
        o_r[0] = h[:, :128] * dinv
        o_r[1] = h[:, 128:] * dinv

    blk2 = pl.BlockSpec((2, BLKP, 128), lambda i: (0, i, 0))
    return pl.pallas_call(
        body,
        grid=(RP // BLKP,),
        in_specs=[
            blk2, blk2, blk2,
            pl.BlockSpec(b1p.shape, lambda i: (0, 0)),
            pl.BlockSpec(W2bd.shape, lambda i: (0, 0)),
        ],
        out_specs=blk2,
        out_shape=jax.ShapeDtypeStruct((2, RP, 128), jnp.float32),
    )(degp, acc1, g1, b1p, W2bd)


def _tc3(degp, acc2, g2, b2p, xp, WLcat, bL1, WL2, bL2):
    def body(d_r, a_r, g_r, b_r, x_r, wc_r, bl1_r, wl2_r, bl2_r, o_r):
        d = d_r[...]
        dinv = lax.rsqrt(d[0] + d[1] + 1.0)
        b = b_r[...]
        ha = jnp.maximum((a_r[0] + g_r[0]) * dinv + b[:, :128], 0.0)
        hb = jnp.maximum((a_r[1] + g_r[1]) * dinv + b[:, 128:], 0.0)
        xT = jnp.transpose(x_r[...])
        haT = jnp.transpose(ha)
        hbT = jnp.transpose(hb)
        dn = (((0,), (0,)), ((), ()))
        for j in range(8):
            rs = slice(LANES * j, LANES * (j + 1))
            lhs = jnp.concatenate([xT[rs], haT[rs], hbT[rs]], axis=0)
            z = lax.dot_general(lhs, wc_r[...], dn,
                                preferred_element_type=jnp.float32)
            z = jnp.maximum(z + bl1_r[...], 0.0)
            o = jnp.dot(z, wl2_r[...], preferred_element_type=jnp.float32)
            o_r[:, pl.ds(j, 1)] = jax.nn.sigmoid(o + bl2_r[...])

    blk2 = pl.BlockSpec((2, BLKP3, 128), lambda i: (0, i, 0))
    full = lambda a: pl.BlockSpec(a.shape, lambda i: (0, 0))
    return pl.pallas_call(
        body,
        grid=(RP // BLKP3,),
        in_specs=[
            blk2, blk2, blk2,
            pl.BlockSpec(b2p.shape, lambda i: (0, 0)),
            pl.BlockSpec((BLKP3, 128), lambda i: (i, 0)),
            full(WLcat), full(bL1), full(WL2), full(bL2),
        ],
        out_specs=pl.BlockSpec((BLKP3, 8), lambda i: (i, 0)),
        out_shape=jax.ShapeDtypeStruct((RP, 8), jnp.float32),
    )(degp, acc2, g2, b2p, xp, WLcat, bL1, WL2, bL2)


def _block_diag8(w):
    k = w.shape[1]
    out = jnp.zeros((128, 8 * k), jnp.float32)
    for j in range(8):
        out = out.at[16 * j:16 * (j + 1), k * j:k * (j + 1)].set(w)
    return out


def kernel(x, edge_index, W1, b1, W2, b2, WL1, bL1, WL2, bL2):
    n_nodes = x.shape[0]
    in_ch = x.shape[1]
    assert n_nodes == N_NODES and edge_index.shape[1] == N_EDGES

    pad = EDGES_PAD - N_EDGES
    pad_src = (jnp.arange(pad, dtype=jnp.int32) * 17) % N_NODES
    pad_dst = N_NODES + (jnp.arange(pad, dtype=jnp.int32) % (NP - N_NODES))
    ei = jnp.concatenate(
        [edge_index.astype(jnp.int32),
         jnp.stack([pad_src, pad_dst])], axis=1).reshape(2, IDX_ROWS, IDX_W)

    xpad = jnp.zeros((NP, LANES), jnp.float32).at[:N_NODES, :in_ch].set(x)
    xp = xpad.reshape(RP, 128)

    W1p = jnp.zeros((LANES, 32), jnp.float32).at[:in_ch].set(W1)
    W1bd = jnp.concatenate(
        [_block_diag8(W1p[:, :16]), _block_diag8(W1p[:, 16:])], axis=1)
    W2bd = jnp.block(
        [[_block_diag8(W2[:16, :16]), _block_diag8(W2[:16, 16:])],
         [_block_diag8(W2[16:, :16]), _block_diag8(W2[16:, 16:])]])
    b1p = jnp.concatenate([jnp.tile(b1[:16], 8), jnp.tile(b1[16:], 8)])
    b1p = b1p.reshape(1, 256)
    b2p = jnp.concatenate([jnp.tile(b2[:16], 8), jnp.tile(b2[16:], 8)])
    b2p = b2p.reshape(1, 256)
    WL1x = jnp.zeros((LANES, 1024), jnp.float32).at[:in_ch].set(WL1[:in_ch])
    WLcat = jnp.concatenate(
        [WL1x, WL1[in_ch:in_ch + 16], WL1[in_ch + 16:in_ch + 32]], axis=0)
    bL1r = bL1.reshape(1, -1)
    bL2r = bL2.reshape(1, -1)

    degp = _deg_pass(ei).reshape(2, RP, 128)

    g1 = _tc1(xp, W1bd, degp)
    acc1 = _conv_pass(g1.reshape(2, NP, LANES), ei).reshape(2, RP, 128)

    g2 = _tc2(degp, acc1, g1, b1p, W2bd)
    acc2 = _conv_pass(g2.reshape(2, NP, LANES), ei).reshape(2, RP, 128)

    out = _tc3(degp, acc2, g2, b2p, xp, WLcat, bL1r, WL2, bL2r)
    return out.reshape(NP, 1)[:N_NODES]


# speedup vs baseline: 38.9783x; 1.0009x over previous
import functools

import jax
import jax.numpy as jnp
from jax import lax
from jax.experimental import pallas as pl
from jax.experimental.pallas import tpu as pltpu
from jax.experimental.pallas import tpu_sc as plsc

N_NODES = 100000
N_EDGES = 1600000
LANES = 16
IDX_W = 128
K_PER_CHUNK = 6
CHUNK = IDX_W * K_PER_CHUNK
EDGES_PAD = 132 * 16 * CHUNK
IDX_ROWS = EDGES_PAD // IDX_W
CONV_CHUNKS = EDGES_PAD // (16 * CHUNK)
DEG_CHUNKS = EDGES_PAD // (32 * CHUNK)
NBUF = 2
ZB = 136
NP = 100096
RP = NP // 8
ZERO_ROWS = NP // 16
BLKP = 736
BLKP3 = 184


def _sc_mesh():
    return plsc.VectorSubcoreMesh(core_axis_name="c", subcore_axis_name="s")


def _fill_rows(ref, n_rows, val):
    v = jnp.full((LANES,), val, jnp.float32)

    def body(i, carry):
        ref[i] = v
        return carry

    lax.fori_loop(0, n_rows, body, 0)


def _zero_acc(acc, zbuf, s):
    base = s * ZERO_ROWS
    nz = zbuf.shape[0]
    done = 0
    while done < ZERO_ROWS:
        step = min(nz, ZERO_ROWS - done)
        pltpu.sync_copy(zbuf.at[pl.ds(0, step)],
                        acc.at[pl.ds(base + done, step)])
        done += step


def _deg_pass(ei):
    @functools.partial(
        pl.kernel,
        out_type=jax.ShapeDtypeStruct((2, NP, LANES), jnp.float32),
        mesh=_sc_mesh(),
        compiler_params=pltpu.CompilerParams(use_tc_tiling_on_sc=False),
        scratch_types=[
            pltpu.VMEM((2, CHUNK), jnp.int32),
            pltpu.VMEM((CHUNK, LANES), jnp.float32),
            pltpu.VMEM((ZB, LANES), jnp.float32),
            pltpu.VMEM_SHARED((NP, LANES), jnp.float32),
            pltpu.SemaphoreType.DMA,
            pltpu.SemaphoreType.DMA,
        ],
    )
    def kdeg(ei_h, out_h, didx, ones_v, zbuf, acc, sem0, sem1):
        c = lax.axis_index("c")
        s = lax.axis_index("s")
        sems = [sem0, sem1]
        _fill_rows(ones_v, CHUNK, 1.0)
        _fill_rows(zbuf, ZB, 0.0)
        _zero_acc(acc, zbuf, s)
        plsc.subcore_barrier()
        wid = s * 2 + c
        base = wid * DEG_CHUNKS

        def stage(b, i):
            rb = base + i
            pltpu.sync_copy(ei_h.at[1, rb], didx.at[b])

        def fire(b):
            pltpu.async_copy(ones_v, acc.at[didx.at[b]], sems[b], add=True)

        def drain(b):
            pltpu.make_async_copy(
                ones_v, acc.at[didx.at[b]], sems[b]).wait()

        stage(0, 0)

        def body(t, carry):
            for k in range(2):
                i = 2 * t + k
                fire(k)

                @pl.when(i >= 1)
                def _():
                    drain(1 - k)

                @pl.when(i + 1 < DEG_CHUNKS)
                def _():
                    stage(1 - k, i + 1)
            return carry

        lax.fori_loop(0, DEG_CHUNKS // 2, body, 0)
        drain(1)
        plsc.subcore_barrier()
        sl = pl.ds(s * ZERO_ROWS, ZERO_ROWS)

        @pl.when(c == 0)
        def _():
            pltpu.sync_copy(acc.at[sl], out_h.at[0, sl])

        @pl.when(c == 1)
        def _():
            pltpu.sync_copy(acc.at[sl], out_h.at[1, sl])

    return kdeg(ei)


def _conv_pass(g, ei):
    @functools.partial(
        pl.kernel,
        out_type=jax.ShapeDtypeStruct((2, NP, LANES), jnp.float32),
        mesh=_sc_mesh(),
        compiler_params=pltpu.CompilerParams(use_tc_tiling_on_sc=False),
        scratch_types=[
            pltpu.VMEM((NBUF, CHUNK), jnp.int32),
            pltpu.VMEM((NBUF, CHUNK), jnp.int32),
            pltpu.VMEM((NBUF, CHUNK, LANES), jnp.float32),
            pltpu.VMEM((ZB, LANES), jnp.float32),
            pltpu.VMEM_SHARED((NP, LANES), jnp.float32),
        ] + [pltpu.SemaphoreType.DMA] * 4,
    )
    def kconv(g_h, ei_h, out_h, sidx, didx, rows, zbuf, acc,
              sg0, sg1, ss0, ss1):
        c = lax.axis_index("c")
        s = lax.axis_index("s")
        semg = [sg0, sg1]
        sems = [ss0, ss1]
        _fill_rows(zbuf, ZB, 0.0)
        _zero_acc(acc, zbuf, s)
        plsc.subcore_barrier()
        base = s * CONV_CHUNKS

        def stage(b, i):
            rb = base + i
            pltpu.sync_copy(ei_h.at[0, rb], sidx.at[b])
            pltpu.sync_copy(ei_h.at[1, rb], didx.at[b])

        def fire_g(b):
            @pl.when(c == 0)
            def _():
                pltpu.async_copy(g_h.at[0].at[sidx.at[b]], rows.at[b],
                                 semg[b])

            @pl.when(c == 1)
            def _():
                pltpu.async_copy(g_h.at[1].at[sidx.at[b]], rows.at[b],
                                 semg[b])

        def wait_g(b):
            pltpu.make_async_copy(
                g_h.at[0].at[sidx.at[b]], rows.at[b], semg[b]).wait()

        def fire_s(b):
            pltpu.async_copy(rows.at[b], acc.at[didx.at[b]], sems[b],
                             add=True)

        def wait_s(b):
            pltpu.make_async_copy(
                rows.at[b], acc.at[didx.at[b]], sems[b]).wait()

        stage(0, 0)
        fire_g(0)

        def body(t, carry):
            for k in range(2):
                i = 2 * t + k
                wait_g(k)
                fire_s(k)

                @pl.when(i >= 1)
                def _():
                    wait_s(1 - k)

                @pl.when(i + 1 < CONV_CHUNKS)
                def _():
                    stage(1 - k, i + 1)
                    fire_g(1 - k)
            return carry

        lax.fori_loop(0, CONV_CHUNKS // 2, body, 0)
        wait_s(1)
        plsc.subcore_barrier()
        sl = pl.ds(s * ZERO_ROWS, ZERO_ROWS)

        @pl.when(c == 0)
        def _():
            pltpu.sync_copy(acc.at[sl], out_h.at[0, sl])

        @pl.when(c == 1)
        def _():
            pltpu.sync_copy(acc.at[sl], out_h.at[1, sl])

    return kconv(g, ei)


def _tc1(xp, W1bd, degp):
    def body(x_r, w_r, d_r, o_r):
        d = d_r[...]
        dinv = lax.rsqrt(d[0] + d[1] + 1.0)
        h = jnp.dot(x_r[...], w_r[...], preferred_element_type=jnp.float32)
        o_r[0] = h[:, :128] * dinv
        o_r[1] = h[:, 128:] * dinv

    return pl.pallas_call(
        body,
        grid=(RP // BLKP,),
        in_specs=[
            pl.BlockSpec((BLKP, 128), lambda i: (i, 0)),
            pl.BlockSpec(W1bd.shape, lambda i: (0, 0)),
            pl.BlockSpec((2, BLKP, 128), lambda i: (0, i, 0)),
        ],
        out_specs=pl.BlockSpec((2, BLKP, 128), lambda i: (0, i, 0)),
        out_shape=jax.ShapeDtypeStruct((2, RP, 128), jnp.float32),
    )(xp, W1bd, degp)


def _tc2(degp, acc1, g1, b1p, W2bd):
    def body(d_r, a_r, g_r, b_r, w_r, o_r):
        d = d_r[...]
        dinv = lax.rsqrt(d[0] + d[1] + 1.0)
        b = b_r[...]
        ra = jnp.maximum((a_r[0] + g_r[0]) * dinv + b[:, :128], 0.0)
        rb = jnp.maximum((a_r[1] + g_r[1]) * dinv + b[:, 128:], 0.0)
        h = jnp.dot(jnp.concatenate([ra, rb], axis=1), w_r[...],
                    preferred_element_type=jnp.float32)
        o_r[0] = h[:, :128] * dinv
        o_r[1] = h[:, 128:] * dinv

    blk2 = pl.BlockSpec((2, BLKP, 128), lambda i: (0, i, 0))
    return pl.pallas_call(
        body,
        grid=(RP // BLKP,),
        in_specs=[
            blk2, blk2, blk2,
            pl.BlockSpec(b1p.shape, lambda i: (0, 0)),
            pl.BlockSpec(W2bd.shape, lambda i: (0, 0)),
        ],
        out_specs=blk2,
        out_shape=jax.ShapeDtypeStruct((2, RP, 128), jnp.float32),
    )(degp, acc1, g1, b1p, W2bd)


def _tc3(degp, acc2, g2, b2p, xp, WLcat, bL1, WL2, bL2):
    def body(d_r, a_r, g_r, b_r, x_r, wc_r, bl1_r, wl2_r, bl2_r, o_r):
        d = d_r[...]
        dinv = lax.rsqrt(d[0] + d[1] + 1.0)
        b = b_r[...]
        ha = jnp.maximum((a_r[0] + g_r[0]) * dinv + b[:, :128], 0.0)
        hb = jnp.maximum((a_r[1] + g_r[1]) * dinv + b[:, 128:], 0.0)
        xT = jnp.transpose(x_r[...])
        haT = jnp.transpose(ha)
        hbT = jnp.transpose(hb)
        dn = (((0,), (0,)), ((), ()))
        for j in range(8):
            rs = slice(LANES * j, LANES * (j + 1))
            lhs = jnp.concatenate([xT[rs], haT[rs], hbT[rs]], axis=0)
            z = lax.dot_general(lhs, wc_r[...], dn,
                                preferred_element_type=jnp.float32)
            z = jnp.maximum(z + bl1_r[...], 0.0)
            o = jnp.dot(z, wl2_r[...], preferred_element_type=jnp.float32)
            o_r[:, pl.ds(j, 1)] = jax.nn.sigmoid(o + bl2_r[...])

    blk2 = pl.BlockSpec((2, BLKP3, 128), lambda i: (0, i, 0))
    full = lambda a: pl.BlockSpec(a.shape, lambda i: (0, 0))
    return pl.pallas_call(
        body,
        grid=(RP // BLKP3,),
        in_specs=[
            blk2, blk2, blk2,
            pl.BlockSpec(b2p.shape, lambda i: (0, 0)),
            pl.BlockSpec((BLKP3, 128), lambda i: (i, 0)),
            full(WLcat), full(bL1), full(WL2), full(bL2),
        ],
        out_specs=pl.BlockSpec((BLKP3, 8), lambda i: (i, 0)),
        out_shape=jax.ShapeDtypeStruct((RP, 8), jnp.float32),
    )(degp, acc2, g2, b2p, xp, WLcat, bL1, WL2, bL2)


def _block_diag8(w):
    k = w.shape[1]
    out = jnp.zeros((128, 8 * k), jnp.float32)
    for j in range(8):
        out = out.at[16 * j:16 * (j + 1), k * j:k * (j + 1)].set(w)
    return out


def kernel(x, edge_index, W1, b1, W2, b2, WL1, bL1, WL2, bL2):
    n_nodes = x.shape[0]
    in_ch = x.shape[1]
    assert n_nodes == N_NODES and edge_index.shape[1] == N_EDGES

    pad = EDGES_PAD - N_EDGES
    pad_src = (jnp.arange(pad, dtype=jnp.int32) * 17) % N_NODES
    pad_dst = N_NODES + (jnp.arange(pad, dtype=jnp.int32) % (NP - N_NODES))
    ei = jnp.concatenate(
        [edge_index.astype(jnp.int32),
         jnp.stack([pad_src, pad_dst])], axis=1).reshape(2, EDGES_PAD // CHUNK, CHUNK)

    xpad = jnp.zeros((NP, LANES), jnp.float32).at[:N_NODES, :in_ch].set(x)
    xp = xpad.reshape(RP, 128)

    W1p = jnp.zeros((LANES, 32), jnp.float32).at[:in_ch].set(W1)
    W1bd = jnp.concatenate(
        [_block_diag8(W1p[:, :16]), _block_diag8(W1p[:, 16:])], axis=1)
    W2bd = jnp.block(
        [[_block_diag8(W2[:16, :16]), _block_diag8(W2[:16, 16:])],
         [_block_diag8(W2[16:, :16]), _block_diag8(W2[16:, 16:])]])
    b1p = jnp.concatenate([jnp.tile(b1[:16], 8), jnp.tile(b1[16:], 8)])
    b1p = b1p.reshape(1, 256)
    b2p = jnp.concatenate([jnp.tile(b2[:16], 8), jnp.tile(b2[16:], 8)])
    b2p = b2p.reshape(1, 256)
    WL1x = jnp.zeros((LANES, 1024), jnp.float32).at[:in_ch].set(WL1[:in_ch])
    WLcat = jnp.concatenate(
        [WL1x, WL1[in_ch:in_ch + 16], WL1[in_ch + 16:in_ch + 32]], axis=0)
    bL1r = bL1.reshape(1, -1)
    bL2r = bL2.reshape(1, -1)

    degp = _deg_pass(ei).reshape(2, RP, 128)

    g1 = _tc1(xp, W1bd, degp)
    acc1 = _conv_pass(g1.reshape(2, NP, LANES), ei).reshape(2, RP, 128)

    g2 = _tc2(degp, acc1, g1, b1p, W2bd)
    acc2 = _conv_pass(g2.reshape(2, NP, LANES), ei).reshape(2, RP, 128)

    out = _tc3(degp, acc2, g2, b2p, xp, WLcat, bL1r, WL2, bL2r)
    return out.reshape(NP, 1)[:N_NODES]
